# baseline (speedup 1.0000x reference)
import jax
import jax.numpy as jnp
from jax import lax
from jax.experimental import pallas as pl
from jax.experimental.pallas import tpu as pltpu

_RB = 8192


def _body(x_ref, pd_ref, cost_col_ref, cost_row_ref, pmax_col_ref,
          pmax_row_ref, w_ref, out_ref, cb_ref):
    n_unit = cost_col_ref.shape[0]

    @pl.when(pl.program_id(0) == 0)
    def _():
        cc = cost_col_ref[...]
        cr = cost_row_ref[...]
        jcol = lax.broadcasted_iota(jnp.int32, (n_unit, n_unit), 0)
        grow = lax.broadcasted_iota(jnp.int32, (n_unit, n_unit), 1)
        before = (cc < cr) | ((cc == cr) & (jcol < grow))
        cb_ref[...] = jnp.sum(
            jnp.where(before, pmax_col_ref[...], 0.0), axis=0, keepdims=True)

    total_d = jnp.sum(pd_ref[...]) - w_ref[0, 0] * x_ref[...]
    out_ref[...] = jnp.clip(total_d - cb_ref[...], 0.0, pmax_row_ref[...])


def kernel(x, Cost, Pd, w_capacity, Pmax):
    B = x.shape[0]
    n_unit = Cost.shape[0]
    x_col = x.reshape(B, 1)
    pd2d = Pd.reshape(-1, 128)
    cost_col = Cost.reshape(n_unit, 1)
    cost_row = Cost.reshape(1, n_unit)
    pmax_col = Pmax.reshape(n_unit, 1)
    pmax_row = Pmax.reshape(1, n_unit)
    w2d = w_capacity.reshape(1, 1)

    grid = (B // _RB,)
    return pl.pallas_call(
        _body,
        grid=grid,
        in_specs=[
            pl.BlockSpec((_RB, 1), lambda i: (i, 0)),
            pl.BlockSpec(pd2d.shape, lambda i: (0, 0)),
            pl.BlockSpec((n_unit, 1), lambda i: (0, 0)),
            pl.BlockSpec((1, n_unit), lambda i: (0, 0)),
            pl.BlockSpec((n_unit, 1), lambda i: (0, 0)),
            pl.BlockSpec((1, n_unit), lambda i: (0, 0)),
            pl.BlockSpec((1, 1), lambda i: (0, 0)),
        ],
        out_specs=pl.BlockSpec((_RB, n_unit), lambda i: (i, 0)),
        out_shape=jax.ShapeDtypeStruct((B, n_unit), jnp.float32),
        scratch_shapes=[pltpu.VMEM((1, n_unit), jnp.float32)],
        compiler_params=pltpu.CompilerParams(
            dimension_semantics=("arbitrary",)),
    )(x_col, pd2d, cost_col, cost_row, pmax_col, pmax_row, w2d)
